# baseline (speedup 1.0000x reference)
import jax
import jax.numpy as jnp
from jax.experimental import pallas as pl
from jax.experimental.pallas import tpu as pltpu


def _dot(a, b):
    return jax.lax.dot_general(a, b, (((1,), (0,)), ((), ())),
                               preferred_element_type=jnp.float32)


def _dot_tn(a, b):
    return jax.lax.dot_general(a, b, (((0,), (0,)), ((), ())),
                               preferred_element_type=jnp.float32)


def _dot_nt(a, b):
    return jax.lax.dot_general(a, b, (((1,), (1,)), ((), ())),
                               preferred_element_type=jnp.float32)


def _body(x_hbm, adj_s_hbm, adj_t_hbm, tw1_hbm, tw2_hbm, tw3_hbm,
          sw1_hbm, sw2_hbm, sw3_hbm, d1w_hbm, d1b_hbm, d2w_hbm, d2b_hbm,
          d3w_hbm, d3b_hbm, fcw_hbm, fcb_hbm, out_hbm,
          x_v, adj_s_v, adj_t_v, tw1_v, tw2_v, tw3_v,
          sw1_v, sw2_v, sw3_v, d1w_v, d1b_v, d2w_v, d2b_v,
          d3w_v, d3b_v, fcw_v, fcb_v, out_v, sems):
    hbm = (x_hbm, tw1_hbm, adj_t_hbm, sw1_hbm, adj_s_hbm, tw2_hbm, tw3_hbm,
           sw2_hbm, sw3_hbm, d1w_hbm, d1b_hbm, d2w_hbm, d2b_hbm, d3w_hbm,
           d3b_hbm, fcw_hbm, fcb_hbm)
    vmem = (x_v, tw1_v, adj_t_v, sw1_v, adj_s_v, tw2_v, tw3_v,
            sw2_v, sw3_v, d1w_v, d1b_v, d2w_v, d2b_v, d3w_v,
            d3b_v, fcw_v, fcb_v)
    copies = [pltpu.make_async_copy(h, v, sems.at[i])
              for i, (h, v) in enumerate(zip(hbm, vmem))]
    for c in copies:
        c.start()
    (c_x, c_tw1, c_adj_t, c_sw1, c_adj_s, c_tw2, c_tw3, c_sw2, c_sw3,
     c_d1w, c_d1b, c_d2w, c_d2b, c_d3w, c_d3b, c_fcw, c_fcb) = copies

    c_x.wait()
    c_tw1.wait()
    x = x_v[...]
    t1 = jax.lax.dot_general(tw1_v[...], x, (((0,), (1,)), ((), ())),
                             preferred_element_type=jnp.float32)
    c_adj_t.wait()
    adj_t = adj_t_v[...]
    h = jnp.maximum(_dot_nt(t1, adj_t), 0.0)
    c_tw2.wait()
    h = jnp.maximum(_dot_nt(_dot_tn(tw2_v[...], h), adj_t), 0.0)
    r = _dot_nt(h, adj_t)
    c_tw3.wait()
    x_t = jnp.maximum(_dot_tn(r, tw3_v[...]), 0.0)

    c_sw1.wait()
    s1 = _dot_tn(sw1_v[...], x)
    c_adj_s.wait()
    adj_s = adj_s_v[...]
    g = jnp.maximum(_dot_nt(s1, adj_s), 0.0)
    c_sw2.wait()
    g = jnp.maximum(_dot_nt(_dot_tn(sw2_v[...], g), adj_s), 0.0)
    q = _dot_nt(g, adj_s)
    c_sw3.wait()
    x_st = jnp.maximum(_dot_tn(sw3_v[...], q), 0.0)

    c_d1w.wait()
    c_d1b.wait()
    c_d2w.wait()
    c_d2b.wait()
    c_d3w.wait()
    c_d3b.wait()
    m23 = _dot(d2w_v[...], d3w_v[...])
    m = _dot(d1w_v[...], m23)
    b_eff = _dot(_dot(d1b_v[...], d2w_v[...]) + d2b_v[...],
                 d3w_v[...]) + d3b_v[...]

    fused = m[0, 0] * x_st + m[1, 0] * x_t + b_eff[0, 0]

    c_fcw.wait()
    c_fcb.wait()
    out_v[...] = _dot_nt(fused, fcw_v[...]) + fcb_v[...]

    c_out = pltpu.make_async_copy(out_v, out_hbm, sems.at[17])
    c_out.start()
    c_out.wait()


def kernel(x, x_adj_s, x_adj_t, t_W1, t_W2, t_W3, s_W1, s_W2, s_W3,
           dec1_W, dec1_b, dec2_W, dec2_b, dec3_W, dec3_b, fc_W, fc_b):
    T, Kd = x.shape
    f32 = jnp.float32
    ins = (x, x_adj_s, x_adj_t,
           t_W1[0], t_W2[0], t_W3[0], s_W1[0], s_W2[0], s_W3[0],
           dec1_W, dec1_b.reshape(1, 8), dec2_W, dec2_b.reshape(1, 4),
           dec3_W, dec3_b.reshape(1, 1), fc_W, fc_b.reshape(1, Kd))
    anyspec = pl.BlockSpec(memory_space=pl.ANY)
    out = pl.pallas_call(
        _body,
        out_shape=jax.ShapeDtypeStruct((T, Kd), f32),
        in_specs=[anyspec] * 17,
        out_specs=anyspec,
        scratch_shapes=[pltpu.VMEM(a.shape, f32) for a in ins]
        + [pltpu.VMEM((T, Kd), f32), pltpu.SemaphoreType.DMA((18,))],
    )(*ins)
    return out
